# baseline (speedup 1.0000x reference)
import functools

import jax
import jax.numpy as jnp
from jax import lax
from jax.experimental import pallas as pl
from jax.experimental.pallas import tpu as pltpu
from jax.experimental.pallas import tpu_sc as plsc

L = 16
NC, NS = 2, 16
B, Q, C = 64, 2048, 256
ROWS = B * Q
RPC = ROWS // NC
CHUNK = 64
SCHUNK = CHUNK * NS
NCHUNK = RPC // SCHUNK
GROUPS = CHUNK // L
JCH = C // L
BIG = 2 ** 30


def _row_maxidx(logv, row):
    vs = [logv[row, pl.ds(L * j, L)] for j in range(JCH)]

    def chain(j0, n):
        m = vs[j0]
        ji = jnp.full((L,), j0, jnp.int32)
        for j in range(j0 + 1, j0 + n):
            gt = vs[j] > m
            m = jnp.maximum(m, vs[j])
            ji = jnp.where(gt, jnp.full((L,), j, jnp.int32), ji)
        return m, ji

    def merge(x, y):
        (mx, jx), (my, jy) = x, y
        return jnp.maximum(mx, my), jnp.where(my > mx, jy, jx)

    c0, c1, c2, c3 = chain(0, 4), chain(4, 4), chain(8, 4), chain(12, 4)
    return merge(merge(c0, c1), merge(c2, c3))


def _group(logv, bxv, labv, mbuf, fbuf, flagv, cntv, g):
    rbase = g * L
    iota = lax.iota(jnp.int32, L)
    for r in range(L):
        m, ji = _row_maxidx(logv, rbase + r)
        fidx = (ji << 4) | iota
        mbuf[pl.ds(r * L, L)] = m
        fbuf[pl.ds(r * L, L)] = fidx

    tidx = iota << 4
    cols = []
    for k in range(L):
        cols.append(plsc.load_gather(mbuf, [tidx + k]))
    gm = cols[0]
    for k in range(1, L):
        gm = jnp.maximum(gm, cols[k])
    cand = jnp.full((L,), BIG, jnp.int32)
    for k in range(L):
        f = plsc.load_gather(fbuf, [tidx + k])
        cand = jnp.minimum(cand, jnp.where(cols[k] == gm, f, jnp.full((L,), BIG, jnp.int32)))

    valid = gm > 0.0
    labels16 = jnp.where(valid, cand, jnp.full((L,), -1, jnp.int32))
    labv[pl.ds(rbase, L)] = labels16
    flags = jnp.where(valid, jnp.full((L,), 1.0, jnp.float32), jnp.full((L,), 0.0, jnp.float32))
    cntv[...] = cntv[...] + flags
    flagv[...] = flags
    qrows = iota >> 2
    qcols = iota & 3
    for q in range(4):
        fl = plsc.load_gather(flagv, [4 * q + qrows])
        rows = rbase + 4 * q + qrows
        b = plsc.load_gather(bxv, [rows, qcols])
        plsc.store_scatter(bxv, [rows, qcols], b * fl)


def _body(logits_hbm, boxes_hbm, labels_hbm, boxesout_hbm, cnt_hbm,
          sh0, sh1, log0, log1, bx0, bx1, lab0, lab1,
          mbuf, fbuf, flagv, cntv, ss0, ss1, sbi0, sbi1, so0, so1):
    cid = lax.axis_index("c")
    sid = lax.axis_index("s")
    wid = sid * NC + cid
    scbase = cid * RPC
    cntv[...] = jnp.zeros((L,), jnp.float32)

    def start_sh(ci, shb, sem):
        @pl.when(sid == 0)
        def _():
            pltpu.async_copy(
                logits_hbm.at[pl.ds(scbase + ci * SCHUNK, SCHUNK), :], shb, sem)

    def wait_sh(shb, sem):
        @pl.when(sid == 0)
        def _():
            pltpu.make_async_copy(
                logits_hbm.at[pl.ds(0, SCHUNK), :], shb, sem).wait()

        plsc.subcore_barrier()

    def myrow(ci):
        return scbase + ci * SCHUNK + sid * CHUNK

    def start_bin(ci, bxb, sem):
        pltpu.async_copy(boxes_hbm.at[pl.ds(myrow(ci), CHUNK), :], bxb, sem)

    def wait_bin(bxb, sem):
        pltpu.make_async_copy(boxes_hbm.at[pl.ds(0, CHUNK), :], bxb, sem).wait()

    def start_out(ci, labb, bxb, sem):
        row = myrow(ci)
        pltpu.async_copy(labb, labels_hbm.at[pl.ds(row, CHUNK)], sem)
        pltpu.async_copy(bxb, boxesout_hbm.at[pl.ds(row, CHUNK), :], sem)

    def wait_out(labb, bxb, sem):
        pltpu.make_async_copy(labb, labels_hbm.at[pl.ds(0, CHUNK)], sem).wait()
        pltpu.make_async_copy(bxb, boxesout_hbm.at[pl.ds(0, CHUNK), :], sem).wait()

    def compute(logb, bxb, labb):
        def g_body(g, c2):
            _group(logb, bxb, labb, mbuf, fbuf, flagv, cntv, g)
            return c2

        lax.fori_loop(0, GROUPS, g_body, 0)

    start_sh(0, sh0, ss0)
    start_sh(1, sh1, ss1)
    start_bin(0, bx0, sbi0)

    def pair(p, carry):
        c0 = 2 * p
        c1 = c0 + 1

        @pl.when(p > 0)
        def _():
            wait_out(lab1, bx1, so1)

        start_bin(c1, bx1, sbi1)

        wait_sh(sh0, ss0)
        pltpu.sync_copy(sh0.at[pl.ds(sid * CHUNK, CHUNK), :], log0)
        plsc.subcore_barrier()
        start_sh(jnp.minimum(c0 + 2, NCHUNK - 1), sh0, ss0)

        wait_bin(bx0, sbi0)
        compute(log0, bx0, lab0)
        start_out(c0, lab0, bx0, so0)

        wait_sh(sh1, ss1)
        pltpu.sync_copy(sh1.at[pl.ds(sid * CHUNK, CHUNK), :], log1)
        plsc.subcore_barrier()
        start_sh(jnp.minimum(c1 + 2, NCHUNK - 1), sh1, ss1)

        wait_bin(bx1, sbi1)
        compute(log1, bx1, lab1)
        start_out(c1, lab1, bx1, so1)

        @pl.when(p < NCHUNK // 2 - 1)
        def _():
            wait_out(lab0, bx0, so0)
            start_bin(c0 + 2, bx0, sbi0)

        return carry

    lax.fori_loop(0, NCHUNK // 2, pair, 0)
    wait_sh(sh0, ss0)
    wait_sh(sh1, ss1)
    wait_out(lab0, bx0, so0)
    wait_out(lab1, bx1, so1)
    pltpu.sync_copy(cntv, cnt_hbm.at[wid])


_sc_call = functools.partial(
    pl.kernel,
    mesh=plsc.VectorSubcoreMesh(core_axis_name="c", subcore_axis_name="s"),
    compiler_params=pltpu.CompilerParams(needs_layout_passes=False),
    out_type=[
        jax.ShapeDtypeStruct((ROWS,), jnp.int32),
        jax.ShapeDtypeStruct((ROWS, 4), jnp.float32),
        jax.ShapeDtypeStruct((NC * NS, L), jnp.float32),
    ],
    scratch_types=[
        pltpu.VMEM_SHARED((SCHUNK, C), jnp.float32),
        pltpu.VMEM_SHARED((SCHUNK, C), jnp.float32),
        pltpu.VMEM((CHUNK, C), jnp.float32),
        pltpu.VMEM((CHUNK, C), jnp.float32),
        pltpu.VMEM((CHUNK, 4), jnp.float32),
        pltpu.VMEM((CHUNK, 4), jnp.float32),
        pltpu.VMEM((CHUNK,), jnp.int32),
        pltpu.VMEM((CHUNK,), jnp.int32),
        pltpu.VMEM((L * L,), jnp.float32),
        pltpu.VMEM((L * L,), jnp.int32),
        pltpu.VMEM((L,), jnp.float32),
        pltpu.VMEM((L,), jnp.float32),
        pltpu.SemaphoreType.DMA,
        pltpu.SemaphoreType.DMA,
        pltpu.SemaphoreType.DMA,
        pltpu.SemaphoreType.DMA,
        pltpu.SemaphoreType.DMA,
        pltpu.SemaphoreType.DMA,
    ],
)(_body)


def kernel(pred_logits, pred_boxes):
    logits = pred_logits.reshape(ROWS, C)
    boxes = pred_boxes.reshape(ROWS, 4)
    labels_flat, boxes_flat, cnts = _sc_call(logits, boxes)
    labels = labels_flat.reshape(B, Q)
    boxes_out = boxes_flat.reshape(B, Q, 4)
    num_boxes = jnp.maximum(jnp.sum(cnts), 1.0)
    return labels, boxes_out, num_boxes


# speedup vs baseline: 1.0240x; 1.0240x over previous
import functools

import jax
import jax.numpy as jnp
from jax import lax
from jax.experimental import pallas as pl
from jax.experimental.pallas import tpu as pltpu
from jax.experimental.pallas import tpu_sc as plsc

L = 16
NC, NS = 2, 16
NW = NC * NS
B, Q, C = 64, 2048, 256
ROWS = B * Q
RPW = ROWS // NW
CHUNK = 128
NCHUNK = RPW // CHUNK
GROUPS = CHUNK // L
JCH = C // L
BIG = 2 ** 30


def _row_maxidx(logv, row):
    vs = [logv[row, pl.ds(L * j, L)] for j in range(JCH)]

    def chain(j0, n):
        m = vs[j0]
        ji = jnp.full((L,), j0, jnp.int32)
        for j in range(j0 + 1, j0 + n):
            gt = vs[j] > m
            m = jnp.maximum(m, vs[j])
            ji = jnp.where(gt, jnp.full((L,), j, jnp.int32), ji)
        return m, ji

    def merge(x, y):
        (mx, jx), (my, jy) = x, y
        return jnp.maximum(mx, my), jnp.where(my > mx, jy, jx)

    c0, c1, c2, c3 = chain(0, 4), chain(4, 4), chain(8, 4), chain(12, 4)
    return merge(merge(c0, c1), merge(c2, c3))


def _group(logv, bxv, labv, mbuf, fbuf, flagv, cntv, g):
    rbase = g * L
    iota = lax.iota(jnp.int32, L)
    for r in range(L):
        m, ji = _row_maxidx(logv, rbase + r)
        fidx = (ji << 4) | iota
        mbuf[pl.ds(r * L, L)] = m
        fbuf[pl.ds(r * L, L)] = fidx

    tidx = iota << 4
    cols = []
    for k in range(L):
        cols.append(plsc.load_gather(mbuf, [tidx + k]))
    gm = cols[0]
    for k in range(1, L):
        gm = jnp.maximum(gm, cols[k])
    cand = jnp.full((L,), BIG, jnp.int32)
    for k in range(L):
        f = plsc.load_gather(fbuf, [tidx + k])
        cand = jnp.minimum(cand, jnp.where(cols[k] == gm, f, jnp.full((L,), BIG, jnp.int32)))

    valid = gm > 0.0
    labels16 = jnp.where(valid, cand, jnp.full((L,), -1, jnp.int32))
    labv[pl.ds(rbase, L)] = labels16
    flags = jnp.where(valid, jnp.full((L,), 1.0, jnp.float32), jnp.full((L,), 0.0, jnp.float32))
    cntv[...] = cntv[...] + flags
    flagv[...] = flags
    qrows = iota >> 2
    qcols = iota & 3
    for q in range(4):
        fl = plsc.load_gather(flagv, [4 * q + qrows])
        rows = rbase + 4 * q + qrows
        b = plsc.load_gather(bxv, [rows, qcols])
        plsc.store_scatter(bxv, [rows, qcols], b * fl)


def _body(logits_hbm, boxes_hbm, labels_hbm, boxesout_hbm, cnt_hbm,
          log0, log1, bx0, bx1, lab0, lab1,
          mbuf, fbuf, flagv, cntv, si0, si1, sbi0, sbi1, so0, so1):
    cid = lax.axis_index("c")
    sid = lax.axis_index("s")
    wid = sid * NC + cid
    row0 = wid * RPW
    cntv[...] = jnp.zeros((L,), jnp.float32)

    def start_login(ci, logb, sem):
        crow = row0 + ci * CHUNK
        pltpu.async_copy(logits_hbm.at[pl.ds(crow, CHUNK), :], logb, sem)

    def wait_login(logb, sem):
        pltpu.make_async_copy(logits_hbm.at[pl.ds(0, CHUNK), :], logb, sem).wait()

    def start_bin(ci, bxb, sem):
        crow = row0 + ci * CHUNK
        pltpu.async_copy(boxes_hbm.at[pl.ds(crow, CHUNK), :], bxb, sem)

    def wait_bin(bxb, sem):
        pltpu.make_async_copy(boxes_hbm.at[pl.ds(0, CHUNK), :], bxb, sem).wait()

    def start_out(ci, labb, bxb, sem):
        crow = row0 + ci * CHUNK
        pltpu.async_copy(labb, labels_hbm.at[pl.ds(crow, CHUNK)], sem)
        pltpu.async_copy(bxb, boxesout_hbm.at[pl.ds(crow, CHUNK), :], sem)

    def wait_out(labb, bxb, sem):
        pltpu.make_async_copy(labb, labels_hbm.at[pl.ds(0, CHUNK)], sem).wait()
        pltpu.make_async_copy(bxb, boxesout_hbm.at[pl.ds(0, CHUNK), :], sem).wait()

    def compute(logb, bxb, labb):
        def g_body(g, c2):
            _group(logb, bxb, labb, mbuf, fbuf, flagv, cntv, g)
            return c2

        lax.fori_loop(0, GROUPS, g_body, 0)

    start_login(0, log0, si0)
    start_bin(0, bx0, sbi0)

    def pair(p, carry):
        c0 = 2 * p

        @pl.when(p > 0)
        def _():
            wait_out(lab1, bx1, so1)

        start_login(c0 + 1, log1, si1)
        start_bin(c0 + 1, bx1, sbi1)
        wait_login(log0, si0)
        wait_bin(bx0, sbi0)
        compute(log0, bx0, lab0)
        start_out(c0, lab0, bx0, so0)
        wait_login(log1, si1)
        wait_bin(bx1, sbi1)
        compute(log1, bx1, lab1)
        start_out(c0 + 1, lab1, bx1, so1)

        @pl.when(p < NCHUNK // 2 - 1)
        def _():
            wait_out(lab0, bx0, so0)
            start_login(c0 + 2, log0, si0)
            start_bin(c0 + 2, bx0, sbi0)

        return carry

    lax.fori_loop(0, NCHUNK // 2, pair, 0)
    wait_out(lab0, bx0, so0)
    wait_out(lab1, bx1, so1)
    pltpu.sync_copy(cntv, cnt_hbm.at[wid])


_sc_call = functools.partial(
    pl.kernel,
    mesh=plsc.VectorSubcoreMesh(core_axis_name="c", subcore_axis_name="s"),
    compiler_params=pltpu.CompilerParams(needs_layout_passes=False),
    out_type=[
        jax.ShapeDtypeStruct((ROWS,), jnp.int32),
        jax.ShapeDtypeStruct((ROWS, 4), jnp.float32),
        jax.ShapeDtypeStruct((NW, L), jnp.float32),
    ],
    scratch_types=[
        pltpu.VMEM((CHUNK, C), jnp.float32),
        pltpu.VMEM((CHUNK, C), jnp.float32),
        pltpu.VMEM((CHUNK, 4), jnp.float32),
        pltpu.VMEM((CHUNK, 4), jnp.float32),
        pltpu.VMEM((CHUNK,), jnp.int32),
        pltpu.VMEM((CHUNK,), jnp.int32),
        pltpu.VMEM((L * L,), jnp.float32),
        pltpu.VMEM((L * L,), jnp.int32),
        pltpu.VMEM((L,), jnp.float32),
        pltpu.VMEM((L,), jnp.float32),
        pltpu.SemaphoreType.DMA,
        pltpu.SemaphoreType.DMA,
        pltpu.SemaphoreType.DMA,
        pltpu.SemaphoreType.DMA,
        pltpu.SemaphoreType.DMA,
        pltpu.SemaphoreType.DMA,
    ],
)(_body)


def kernel(pred_logits, pred_boxes):
    logits = pred_logits.reshape(ROWS, C)
    boxes = pred_boxes.reshape(ROWS, 4)
    labels_flat, boxes_flat, cnts = _sc_call(logits, boxes)
    labels = labels_flat.reshape(B, Q)
    boxes_out = boxes_flat.reshape(B, Q, 4)
    num_boxes = jnp.maximum(jnp.sum(cnts), 1.0)
    return labels, boxes_out, num_boxes


# speedup vs baseline: 1.3043x; 1.2737x over previous
import functools

import jax
import jax.numpy as jnp
from jax import lax
from jax.experimental import pallas as pl
from jax.experimental.pallas import tpu as pltpu
from jax.experimental.pallas import tpu_sc as plsc

L = 16
NC, NS = 2, 16
NW = NC * NS
B, Q, C = 64, 2048, 256
ROWS = B * Q
RPW = ROWS // NW
CHUNK = 64
NCHUNK = RPW // CHUNK
GROUPS = CHUNK // L
JCH = C // L
BIG = 2 ** 30


def _row_maxidx(logv, row):
    vs = [logv[row, pl.ds(L * j, L)] for j in range(JCH)]

    def chain(j0, n):
        m = vs[j0]
        ji = jnp.full((L,), j0, jnp.int32)
        for j in range(j0 + 1, j0 + n):
            gt = vs[j] > m
            m = jnp.maximum(m, vs[j])
            ji = jnp.where(gt, jnp.full((L,), j, jnp.int32), ji)
        return m, ji

    def merge(x, y):
        (mx, jx), (my, jy) = x, y
        return jnp.maximum(mx, my), jnp.where(my > mx, jy, jx)

    c0, c1, c2, c3 = chain(0, 4), chain(4, 4), chain(8, 4), chain(12, 4)
    return merge(merge(c0, c1), merge(c2, c3))


def _group(logv, bxv, bxov, labv, mbuf, fbuf, flagv, cntv, g):
    rbase = g * L
    iota = lax.iota(jnp.int32, L)
    for r in range(L):
        m, ji = _row_maxidx(logv, rbase + r)
        fidx = (ji << 4) | iota
        mbuf[pl.ds(r * L, L)] = m
        fbuf[pl.ds(r * L, L)] = fidx

    tidx = iota << 4
    cols = []
    for k in range(L):
        cols.append(plsc.load_gather(mbuf, [tidx + k]))
    gm = cols[0]
    for k in range(1, L):
        gm = jnp.maximum(gm, cols[k])
    cand = jnp.full((L,), BIG, jnp.int32)
    for k in range(L):
        f = plsc.load_gather(fbuf, [tidx + k])
        cand = jnp.minimum(cand, jnp.where(cols[k] == gm, f, jnp.full((L,), BIG, jnp.int32)))

    valid = gm > 0.0
    labels16 = jnp.where(valid, cand, jnp.full((L,), -1, jnp.int32))
    labv[pl.ds(rbase, L)] = labels16
    flags = jnp.where(valid, jnp.full((L,), 1.0, jnp.float32), jnp.full((L,), 0.0, jnp.float32))
    cntv[...] = cntv[...] + flags
    flagv[...] = flags
    qrows = iota >> 2
    qcols = iota & 3
    for q in range(4):
        fl = plsc.load_gather(flagv, [4 * q + qrows])
        rows = rbase + 4 * q + qrows
        b = plsc.load_gather(bxv, [rows, qcols])
        plsc.store_scatter(bxov, [rows, qcols], b * fl)


def _body(logits_hbm, boxes_hbm, labels_hbm, boxesout_hbm, cnt_hbm,
          log0, log1, bx0, bx1, bxo0, bxo1, lab0, lab1,
          mbuf, fbuf, flagv, cntv, si0, si1, so0, so1):
    cid = lax.axis_index("c")
    sid = lax.axis_index("s")
    wid = sid * NC + cid
    row0 = wid * RPW
    cntv[...] = jnp.zeros((L,), jnp.float32)

    def start_in(ci, logb, bxb, sem):
        crow = row0 + ci * CHUNK
        pltpu.async_copy(logits_hbm.at[pl.ds(crow, CHUNK), :], logb, sem)
        pltpu.async_copy(boxes_hbm.at[pl.ds(crow, CHUNK), :], bxb, sem)

    def wait_in(logb, bxb, sem):
        pltpu.make_async_copy(logits_hbm.at[pl.ds(0, CHUNK), :], logb, sem).wait()
        pltpu.make_async_copy(boxes_hbm.at[pl.ds(0, CHUNK), :], bxb, sem).wait()

    def start_out(ci, labb, bxob, sem):
        crow = row0 + ci * CHUNK
        pltpu.async_copy(labb, labels_hbm.at[pl.ds(crow, CHUNK)], sem)
        pltpu.async_copy(bxob, boxesout_hbm.at[pl.ds(crow, CHUNK), :], sem)

    def wait_out(labb, bxob, sem):
        pltpu.make_async_copy(labb, labels_hbm.at[pl.ds(0, CHUNK)], sem).wait()
        pltpu.make_async_copy(bxob, boxesout_hbm.at[pl.ds(0, CHUNK), :], sem).wait()

    def compute(logb, bxb, bxob, labb):
        def g_body(g, c2):
            _group(logb, bxb, bxob, labb, mbuf, fbuf, flagv, cntv, g)
            return c2

        lax.fori_loop(0, GROUPS, g_body, 0)

    start_in(0, log0, bx0, si0)

    def pair(p, carry):
        c0 = 2 * p
        start_in(c0 + 1, log1, bx1, si1)
        wait_in(log0, bx0, si0)

        @pl.when(p > 0)
        def _():
            wait_out(lab0, bxo0, so0)

        compute(log0, bx0, bxo0, lab0)
        start_out(c0, lab0, bxo0, so0)
        start_in(jnp.minimum(c0 + 2, NCHUNK - 1), log0, bx0, si0)
        wait_in(log1, bx1, si1)

        @pl.when(p > 0)
        def _():
            wait_out(lab1, bxo1, so1)

        compute(log1, bx1, bxo1, lab1)
        start_out(c0 + 1, lab1, bxo1, so1)
        return carry

    lax.fori_loop(0, NCHUNK // 2, pair, 0)
    wait_in(log0, bx0, si0)
    wait_out(lab0, bxo0, so0)
    wait_out(lab1, bxo1, so1)
    pltpu.sync_copy(cntv, cnt_hbm.at[wid])


_sc_call = functools.partial(
    pl.kernel,
    mesh=plsc.VectorSubcoreMesh(core_axis_name="c", subcore_axis_name="s"),
    compiler_params=pltpu.CompilerParams(needs_layout_passes=False),
    out_type=[
        jax.ShapeDtypeStruct((ROWS,), jnp.int32),
        jax.ShapeDtypeStruct((ROWS, 4), jnp.float32),
        jax.ShapeDtypeStruct((NW, L), jnp.float32),
    ],
    scratch_types=[
        pltpu.VMEM((CHUNK, C), jnp.float32),
        pltpu.VMEM((CHUNK, C), jnp.float32),
        pltpu.VMEM((CHUNK, 4), jnp.float32),
        pltpu.VMEM((CHUNK, 4), jnp.float32),
        pltpu.VMEM((CHUNK, 4), jnp.float32),
        pltpu.VMEM((CHUNK, 4), jnp.float32),
        pltpu.VMEM((CHUNK,), jnp.int32),
        pltpu.VMEM((CHUNK,), jnp.int32),
        pltpu.VMEM((L * L,), jnp.float32),
        pltpu.VMEM((L * L,), jnp.int32),
        pltpu.VMEM((L,), jnp.float32),
        pltpu.VMEM((L,), jnp.float32),
        pltpu.SemaphoreType.DMA,
        pltpu.SemaphoreType.DMA,
        pltpu.SemaphoreType.DMA,
        pltpu.SemaphoreType.DMA,
    ],
)(_body)


def kernel(pred_logits, pred_boxes):
    logits = pred_logits.reshape(ROWS, C)
    boxes = pred_boxes.reshape(ROWS, 4)
    labels_flat, boxes_flat, cnts = _sc_call(logits, boxes)
    labels = labels_flat.reshape(B, Q)
    boxes_out = boxes_flat.reshape(B, Q, 4)
    num_boxes = jnp.maximum(jnp.sum(cnts), 1.0)
    return labels, boxes_out, num_boxes
